# baseline (speedup 1.0000x reference)
import jax
import jax.numpy as jnp
from jax import lax
from jax.experimental import pallas as pl
from jax.experimental.pallas import tpu as pltpu
from jax.experimental.pallas import tpu_sc as plsc

_N_MEM = 16
_L = 16
_B_PER_W = 32
_N_INPUTS = 128
_N_RULES = 512
_POS = _N_INPUTS * _N_MEM
_CSTRIDE = _POS + 1
_OSTRIDE = _N_RULES + 1


def _splat(v, l):
    return jnp.take(v, jnp.full((_L,), l, jnp.int32))


def _sc_body(fx_hbm, selt_hbm, out_hbm, chunk_v, sel_v, out_v):
    wid = lax.axis_index("s") * 2 + lax.axis_index("c")
    b0 = wid * _B_PER_W
    pltpu.sync_copy(fx_hbm.at[pl.ds(b0, _B_PER_W), :],
                    chunk_v.at[:, pl.ds(0, _POS)])
    lane = lax.iota(jnp.int32, _L)
    ones = jnp.full((_L,), 1.0, jnp.float32)
    plsc.store_scatter(chunk_v, [lane, jnp.full((_L,), _POS, jnp.int32)], ones)
    plsc.store_scatter(chunk_v, [lane + _L, jnp.full((_L,), _POS, jnp.int32)],
                       ones)
    half = jnp.full((_L,), 0.5, jnp.float32)
    row_lo = lane
    row_hi = lane + _L

    def quarter_body(q, carry):
        pltpu.sync_copy(selt_hbm.at[pl.ds(q * 128, 128), :], sel_v)

        def rule_body(rl, carry2):
            pvecs = []
            for g in range(8):
                y = sel_v[rl, pl.ds(g * _L, _L)] * jnp.float32(_N_MEM)
                f = y.astype(jnp.int32)
                frac = y - f.astype(jnp.float32)
                m = (f + jnp.where(frac > half, 1, 0)
                     + jnp.where(frac == half, f & 1, 0))
                i_base = (lane + g * _L) * _N_MEM
                pvecs.append(jnp.where(m < _N_MEM, i_base + m, _POS))
            acc = [jnp.full((_L,), 1.0, jnp.float32) for _ in range(8)]
            for g in range(8):
                for l in range(_L):
                    col = _splat(pvecs[g], l)
                    glo = plsc.load_gather(chunk_v, [row_lo, col])
                    ghi = plsc.load_gather(chunk_v, [row_hi, col])
                    k = l % 4
                    acc[k] = acc[k] * glo
                    acc[4 + k] = acc[4 + k] * ghi
            lo = (acc[0] * acc[1]) * (acc[2] * acc[3])
            hi = (acc[4] * acc[5]) * (acc[6] * acc[7])
            rcol = jnp.full((_L,), q * 128 + rl, jnp.int32)
            plsc.store_scatter(out_v, [row_lo, rcol], lo)
            plsc.store_scatter(out_v, [row_hi, rcol], hi)
            return carry2

        lax.fori_loop(0, 128, rule_body, 0)
        return carry

    lax.fori_loop(0, _N_RULES // 128, quarter_body, 0)
    pltpu.sync_copy(out_v.at[:, pl.ds(0, _N_RULES)],
                    out_hbm.at[pl.ds(b0, _B_PER_W), :])


def kernel(fuzzified_x, input_selectors):
    b = fuzzified_x.shape[0]
    fx2 = fuzzified_x.reshape(b, _POS)
    sel_t = input_selectors.T
    mesh = plsc.VectorSubcoreMesh(core_axis_name="c", subcore_axis_name="s")
    f = pl.kernel(
        _sc_body,
        out_type=jax.ShapeDtypeStruct((b, _N_RULES), jnp.float32),
        mesh=mesh,
        compiler_params=pltpu.CompilerParams(needs_layout_passes=False),
        scratch_types=[
            pltpu.VMEM((_B_PER_W, _CSTRIDE), jnp.float32),
            pltpu.VMEM((128, _N_INPUTS), jnp.float32),
            pltpu.VMEM((_B_PER_W, _OSTRIDE), jnp.float32),
        ],
    )
    return f(fx2, sel_t)


# speedup vs baseline: 3.4926x; 3.4926x over previous
import jax
import jax.numpy as jnp
from jax import lax
from jax.experimental import pallas as pl
from jax.experimental.pallas import tpu as pltpu
from jax.experimental.pallas import tpu_sc as plsc

_N_MEM = 16
_L = 16
_B_PER_W = 32
_N_INPUTS = 128
_N_RULES = 512
_POS = _N_INPUTS * _N_MEM
_CSTRIDE = _POS + 8
_OSTRIDE = _N_RULES + 8


def _splat(v, l):
    return jnp.take(v, jnp.full((_L,), l, jnp.int32))


def _sc_body(fx_hbm, selt_hbm, out_hbm, chunk_v, sel_v, out_v):
    wid = lax.axis_index("s") * 2 + lax.axis_index("c")
    b0 = wid * _B_PER_W
    for r in range(_B_PER_W):
        pltpu.sync_copy(fx_hbm.at[pl.ds((b0 + r) * _POS, _POS)],
                        chunk_v.at[pl.ds(r * _CSTRIDE, _POS)])
    lane = lax.iota(jnp.int32, _L)
    ones = jnp.full((_L,), 1.0, jnp.float32)
    plsc.store_scatter(chunk_v, [lane * _CSTRIDE + _POS], ones)
    plsc.store_scatter(chunk_v, [(lane + _L) * _CSTRIDE + _POS], ones)
    half = jnp.full((_L,), 0.5, jnp.float32)
    rb_lo = lane * _CSTRIDE
    rb_hi = (lane + _L) * _CSTRIDE
    ob_lo = lane * _OSTRIDE
    ob_hi = (lane + _L) * _OSTRIDE

    def quarter_body(q, carry):
        pltpu.sync_copy(selt_hbm.at[pl.ds(q * 128, 128), :], sel_v)

        def rule_body(rl, carry2):
            pvecs = []
            for g in range(8):
                y = sel_v[rl, pl.ds(g * _L, _L)] * jnp.float32(_N_MEM)
                f = y.astype(jnp.int32)
                frac = y - f.astype(jnp.float32)
                m = (f + jnp.where(frac > half, 1, 0)
                     + jnp.where(frac == half, f & 1, 0))
                i_base = (lane + g * _L) * _N_MEM
                pvecs.append(jnp.where(m < _N_MEM, i_base + m, _POS))
            acc = [jnp.full((_L,), 1.0, jnp.float32) for _ in range(8)]
            for g in range(8):
                for l in range(_L):
                    col = _splat(pvecs[g], l)
                    glo = plsc.load_gather(chunk_v, [rb_lo + col])
                    ghi = plsc.load_gather(chunk_v, [rb_hi + col])
                    k = l % 4
                    acc[k] = acc[k] * glo
                    acc[4 + k] = acc[4 + k] * ghi
            lo = (acc[0] * acc[1]) * (acc[2] * acc[3])
            hi = (acc[4] * acc[5]) * (acc[6] * acc[7])
            r_idx = jnp.full((_L,), q * 128 + rl, jnp.int32)
            plsc.store_scatter(out_v, [ob_lo + r_idx], lo)
            plsc.store_scatter(out_v, [ob_hi + r_idx], hi)
            return carry2

        lax.fori_loop(0, 128, rule_body, 0)
        return carry

    lax.fori_loop(0, _N_RULES // 128, quarter_body, 0)
    for r in range(_B_PER_W):
        pltpu.sync_copy(out_v.at[pl.ds(r * _OSTRIDE, _N_RULES)],
                        out_hbm.at[pl.ds((b0 + r) * _N_RULES, _N_RULES)])


def kernel(fuzzified_x, input_selectors):
    b = fuzzified_x.shape[0]
    fx_flat = fuzzified_x.reshape(b * _POS)
    sel_t = input_selectors.T
    mesh = plsc.VectorSubcoreMesh(core_axis_name="c", subcore_axis_name="s")
    f = pl.kernel(
        _sc_body,
        out_type=jax.ShapeDtypeStruct((b * _N_RULES,), jnp.float32),
        mesh=mesh,
        compiler_params=pltpu.CompilerParams(needs_layout_passes=False),
        scratch_types=[
            pltpu.VMEM((_B_PER_W * _CSTRIDE,), jnp.float32),
            pltpu.VMEM((128, _N_INPUTS), jnp.float32),
            pltpu.VMEM((_B_PER_W * _OSTRIDE,), jnp.float32),
        ],
    )
    return f(fx_flat, sel_t).reshape(b, _N_RULES)
